# baseline (speedup 1.0000x reference)
import functools

import jax
import jax.numpy as jnp
from jax import lax
from jax.experimental import pallas as pl
from jax.experimental.pallas import tpu as pltpu
from jax.experimental.pallas import tpu_sc as plsc

VOXEL_SIZE_Z = 0.04
MAX_OFFSET = 5.0

_N_IMG = 8
_C = 128
_H = 224
_W = 384
_HW = _H * _W
_NVOX = 25600
_ROWS = 200
_ZROW = _N_IMG * _HW
_TROWS = _ZROW + 8

_NC = 2
_NS = 16

_CHUNK = 6400
_SEC = 32 * _W
_PXW = 256
_GB = 128
_B1_ON = 1
_B2_ON = 1


def _tc_prologue_body(x_ref, y_ref, z_ref, idx_ref):
    x = x_ref[...]
    y = y_ref[...]
    z = z_ref[...]
    xi = jnp.round(x).astype(jnp.int32)
    yi = jnp.round(y).astype(jnp.int32)
    valid0 = (xi >= 0) & (yi >= 0) & (xi < _W) & (yi < _H) & (z > 0)
    xc = jnp.clip(xi, 0, _W - 1)
    yc = jnp.clip(yi, 0, _H - 1)
    idx_ref[...] = jnp.where(valid0, yc * _W + xc, -1)


def _tc_prologue(xoff, yoff, z):
    out = pl.pallas_call(
        _tc_prologue_body,
        out_shape=jax.ShapeDtypeStruct((_N_IMG * _ROWS, 128), jnp.int32),
    )(xoff, yoff, z)
    return out.reshape(_N_IMG * _NVOX)


def _reduce_max16(vec):
    return lax.rev(lax.sort(vec, dimension=0), (0,))[0]


def _sc_body(feat_hbm, d_hbm, idx_hbm, z_hbm, pts_hbm,
             vol_hbm, validf_hbm, pts3_hbm, t_hbm,
             plane, idxb, zb, iob, src, slab, gidx, rows, sem):
    s = lax.axis_index("s")
    c = lax.axis_index("c")
    wid = c * _NS + s
    img = wid // 4
    cb = wid % 4

    ibase = pl.multiple_of(img * _NVOX, 8)
    ib86 = pl.multiple_of(img * _HW, 8)

    pltpu.sync_copy(idx_hbm.at[pl.ds(ibase, _NVOX)], idxb)

    def maxbody(g, mv):
        return jnp.maximum(mv, idxb[pl.ds(g * 16, 16)])
    mv16 = lax.fori_loop(0, _NVOX // 16, maxbody,
                         jnp.full((16,), -1, jnp.int32))
    maxv = _reduce_max16(mv16)
    nrows0 = (maxv + _W) // _W
    nsec = (nrows0 + 31) // 32

    for ch in range(_NVOX // _CHUNK):
        pltpu.sync_copy(z_hbm.at[pl.ds(ibase + ch * _CHUNK, _CHUNK)], zb)

        def secbody(sec, _, ch=ch):
            lo = sec * _SEC
            pltpu.sync_copy(d_hbm.at[pl.ds(ib86 + lo, _SEC)], plane)

            def wbody(g, _, ch=ch):
                ic = idxb[pl.ds(ch * _CHUNK + g * 16, 16)]
                insec = (ic >= lo) & (ic < lo + _SEC)
                gi = jnp.clip(ic - lo, 0, _SEC - 1)
                dg = plsc.load_gather(plane, [gi])
                zz = zb[pl.ds(g * 16, 16)]
                win = (zz > dg - jnp.float32(VOXEL_SIZE_Z)) & \
                      (zz < dg + jnp.float32(VOXEL_SIZE_Z))
                conv = insec & win
                newv = jnp.where(conv, -(ib86 + ic) - 2, ic)
                idxb[pl.ds(ch * _CHUNK + g * 16, 16)] = newv
                return 0

            lax.fori_loop(0, _CHUNK // 16, wbody, 0)
            return 0

        lax.fori_loop(0, nsec, secbody, 0)

    for ch in range(_NVOX // _CHUNK):
        def fbody(g, _, ch=ch):
            ic = idxb[pl.ds(ch * _CHUNK + g * 16, 16)]
            rowv = jnp.where(ic <= -2, -ic - 2, jnp.int32(_ZROW))
            idxb[pl.ds(ch * _CHUNK + g * 16, 16)] = rowv
            iob[pl.ds(g * 16, 16)] = (rowv != _ZROW).astype(jnp.float32)
            return 0

        lax.fori_loop(0, _CHUNK // 16, fbody, 0)

        @pl.when(cb == 0)
        def _(ch=ch):
            pltpu.sync_copy(iob.at[pl.ds(0, _CHUNK)],
                            validf_hbm.at[pl.ds(ibase + ch * _CHUNK, _CHUNK)])

    for j in range(3):
        @pl.when(cb == j + 1)
        def _(j=j):
            for ch in range(_NVOX // _CHUNK):
                pltpu.sync_copy(
                    pts_hbm.at[pl.ds(j * _NVOX + ch * _CHUNK, _CHUNK)], zb)

                def pbody(g, _, ch=ch):
                    rowv = idxb[pl.ds(ch * _CHUNK + g * 16, 16)]
                    vf = (rowv != _ZROW).astype(jnp.float32)
                    iob[pl.ds(g * 16, 16)] = zb[pl.ds(g * 16, 16)] * vf
                    return 0

                lax.fori_loop(0, _CHUNK // 16, pbody, 0)
                pltpu.sync_copy(
                    iob.at[pl.ds(0, _CHUNK)],
                    pts3_hbm.at[pl.ds((img * 3 + j) * _NVOX + ch * _CHUNK,
                                      _CHUNK)])

    zero16 = jnp.zeros((16,), jnp.float32)

    @pl.when(s == 0)
    def _():
        def zb_(r, _):
            def zl(l, _):
                slab[r, pl.ds(l * 16, 16)] = zero16
                return 0
            lax.fori_loop(0, 8, zl, 0)
            return 0
        lax.fori_loop(0, 8, zb_, 0)
        pltpu.sync_copy(slab.at[pl.ds(0, 8), :],
                        t_hbm.at[pl.ds(_ZROW, 8), :])

    totpx = nrows0 * _W
    iota16 = lax.iota(jnp.int32, 16)

    def b1chunk(k, _):
        px0 = (k * 4 + cb) * _PXW
        for c32 in range(4):
            cps = []
            for cc in range(32):
                cidx = c32 * 32 + cc
                cp = pltpu.async_copy(
                    feat_hbm.at[pl.ds((img * _C + cidx) * _HW + px0, _PXW)],
                    src.at[cc], sem)
                cps.append(cp)
            for cp in cps:
                cp.wait()

            def tr(cc, _, c32=c32):
                lane = jnp.full((16,), c32 * 32, jnp.int32) + cc

                def trg(g, _, cc=cc):
                    v = src[cc, pl.ds(g * 16, 16)]
                    plsc.store_scatter(slab, [g * 16 + iota16, lane], v)
                    return 0

                lax.fori_loop(0, _PXW // 16, trg, 0)
                return 0

            lax.fori_loop(0, 32, tr, 0)
        pltpu.sync_copy(slab, t_hbm.at[pl.ds(ib86 + px0, _PXW), :])
        return 0

    nchunk_all = (totpx + _PXW - 1) // _PXW
    nchunk_my = (nchunk_all - cb + 3) // 4
    lax.fori_loop(0, _B1_ON * nchunk_my, b1chunk, 0)

    plsc.subcore_barrier()

    vbase = pl.multiple_of(ibase + cb * _CHUNK, 8)

    def b2(b, _):
        def cpidx(g, _):
            gidx[pl.ds(g * 16, 16)] = \
                idxb[pl.ds(cb * _CHUNK + b * _GB + g * 16, 16)]
            return 0

        lax.fori_loop(0, _GB // 16, cpidx, 0)
        pltpu.async_copy(t_hbm.at[gidx], rows, sem).wait()
        pltpu.sync_copy(rows, vol_hbm.at[pl.ds(vbase + b * _GB, _GB), :])
        return 0

    lax.fori_loop(0, _B2_ON * (_CHUNK // _GB), b2, 0)


def _sc_gather(feat, dres, idxc, z, pts):
    mesh = plsc.VectorSubcoreMesh(core_axis_name="c", subcore_axis_name="s")
    f = pl.kernel(
        _sc_body,
        out_type=[
            jax.ShapeDtypeStruct((_N_IMG * _NVOX, 128), jnp.float32),
            jax.ShapeDtypeStruct((_N_IMG * _NVOX,), jnp.float32),
            jax.ShapeDtypeStruct((_N_IMG * 3 * _NVOX,), jnp.float32),
            jax.ShapeDtypeStruct((_TROWS, 128), jnp.float32),
        ],
        mesh=mesh,
        scratch_types=[
            pltpu.VMEM((_SEC,), jnp.float32),
            pltpu.VMEM((_NVOX,), jnp.int32),
            pltpu.VMEM((_CHUNK,), jnp.float32),
            pltpu.VMEM((_CHUNK,), jnp.float32),
            pltpu.VMEM((32, _PXW), jnp.float32),
            pltpu.VMEM((_PXW, 128), jnp.float32),
            pltpu.VMEM((_GB,), jnp.int32),
            pltpu.VMEM((_GB, 128), jnp.float32),
            pltpu.SemaphoreType.DMA,
        ],
        compiler_params=pltpu.CompilerParams(needs_layout_passes=False),
    )
    vol, validf, pts3, _ = f(feat, dres, idxc, z, pts)
    return vol, validf, pts3


def kernel(features, points, projection, depth, offsets):
    n, C, H, W = features.shape
    nx, ny, nz = points.shape[-3:]

    feat = features.reshape(n * C * H * W)
    dres = jax.image.resize(depth[:, None, :, :], (n, 1, H, W),
                            method="bilinear")[:, 0].reshape(n * H * W)

    off = jnp.tanh(offsets) * MAX_OFFSET
    off = jnp.broadcast_to(off, (n, off.shape[1], 2))
    ptsb = points.reshape(1, 3, _NVOX)
    ptsb = jnp.broadcast_to(ptsb, (n, 3, _NVOX))
    pts_h = jnp.concatenate(
        [ptsb, jnp.ones((n, 1, _NVOX), dtype=ptsb.dtype)], axis=1)
    p23 = jnp.einsum('bij,bjn->bin', projection, pts_h)
    xoff = p23[:, 0] / p23[:, 2] + off[:, :, 0]
    yoff = p23[:, 1] / p23[:, 2] + off[:, :, 1]
    z = p23[:, 2]

    idxc = _tc_prologue(xoff.reshape(_N_IMG * _ROWS, 128),
                        yoff.reshape(_N_IMG * _ROWS, 128),
                        z.reshape(_N_IMG * _ROWS, 128))
    pts = points.reshape(3 * _NVOX)
    vol2d, validf, pts3 = _sc_gather(feat, dres, idxc, z.reshape(-1), pts)

    volume = jnp.transpose(vol2d.reshape(n, nx, ny, nz, C), (0, 4, 1, 2, 3))
    valid_r = (validf > 0).reshape(n, 1, nx, ny, nz)
    pts3 = pts3.reshape(n, 3, nx, ny, nz)
    return volume, valid_r, pts3


# speedup vs baseline: 11.3757x; 11.3757x over previous
import functools

import jax
import jax.numpy as jnp
from jax import lax
from jax.experimental import pallas as pl
from jax.experimental.pallas import tpu as pltpu
from jax.experimental.pallas import tpu_sc as plsc

VOXEL_SIZE_Z = 0.04
MAX_OFFSET = 5.0

_N_IMG = 8
_C = 128
_H = 224
_W = 384
_HW = _H * _W
_NVOX = 25600
_ROWS = 200
_ZROW = _N_IMG * _HW
_TROWS = _ZROW

_NC = 2
_NS = 16

_CHUNK = 6400
_SEC = 32 * _W
_PXW = 256
_ZB = 16384
_B1_ON = 1
_B2_ON = 1


def _tc_prologue_body(x_ref, y_ref, z_ref, idx_ref):
    x = x_ref[...]
    y = y_ref[...]
    z = z_ref[...]
    xi = jnp.round(x).astype(jnp.int32)
    yi = jnp.round(y).astype(jnp.int32)
    valid0 = (xi >= 0) & (yi >= 0) & (xi < _W) & (yi < _H) & (z > 0)
    xc = jnp.clip(xi, 0, _W - 1)
    yc = jnp.clip(yi, 0, _H - 1)
    idx_ref[...] = jnp.where(valid0, yc * _W + xc, -1)


def _tc_prologue(xoff, yoff, z):
    out = pl.pallas_call(
        _tc_prologue_body,
        out_shape=jax.ShapeDtypeStruct((_N_IMG * _ROWS, 128), jnp.int32),
    )(xoff, yoff, z)
    return out.reshape(_N_IMG * _NVOX)


def _reduce_max16(vec):
    return lax.rev(lax.sort(vec, dimension=0), (0,))[0]


def _sc_body(feat_hbm, d_hbm, idx_hbm, z_hbm, pts_hbm,
             vol_hbm, validf_hbm, pts3_hbm, t_hbm,
             plane, idxb, zb, iob, src, slab, gidxc, vrowc, zbuf, sem):
    s = lax.axis_index("s")
    c = lax.axis_index("c")
    wid = c * _NS + s
    img = wid // 4
    cb = wid % 4

    ibase = pl.multiple_of(img * _NVOX, 8)
    ib86 = pl.multiple_of(img * _HW, 8)

    pltpu.sync_copy(idx_hbm.at[pl.ds(ibase, _NVOX)], idxb)

    def maxbody(g, mv):
        return jnp.maximum(mv, idxb[pl.ds(g * 16, 16)])
    mv16 = lax.fori_loop(0, _NVOX // 16, maxbody,
                         jnp.full((16,), -1, jnp.int32))
    maxv = _reduce_max16(mv16)
    nrows0 = (maxv + _W) // _W
    nsec = (nrows0 + 31) // 32

    for ch in range(_NVOX // _CHUNK):
        pltpu.sync_copy(z_hbm.at[pl.ds(ibase + ch * _CHUNK, _CHUNK)], zb)

        def secbody(sec, _, ch=ch):
            lo = sec * _SEC
            pltpu.sync_copy(d_hbm.at[pl.ds(ib86 + lo, _SEC)], plane)

            def wbody(g, _, ch=ch):
                ic = idxb[pl.ds(ch * _CHUNK + g * 16, 16)]
                insec = (ic >= lo) & (ic < lo + _SEC)
                gi = jnp.clip(ic - lo, 0, _SEC - 1)
                dg = plsc.load_gather(plane, [gi])
                zz = zb[pl.ds(g * 16, 16)]
                win = (zz > dg - jnp.float32(VOXEL_SIZE_Z)) & \
                      (zz < dg + jnp.float32(VOXEL_SIZE_Z))
                conv = insec & win
                newv = jnp.where(conv, -(ib86 + ic) - 2, ic)
                idxb[pl.ds(ch * _CHUNK + g * 16, 16)] = newv
                return 0

            lax.fori_loop(0, _CHUNK // 16, wbody, 0)
            return 0

        lax.fori_loop(0, nsec, secbody, 0)

    iota16 = lax.iota(jnp.int32, 16)
    for ch in range(_NVOX // _CHUNK):
        def fbody(g, _, ch=ch):
            ic = idxb[pl.ds(ch * _CHUNK + g * 16, 16)]
            rowv = jnp.where(ic <= -2, -ic - 2, jnp.int32(_ZROW))
            idxb[pl.ds(ch * _CHUNK + g * 16, 16)] = rowv
            iob[pl.ds(g * 16, 16)] = (rowv != _ZROW).astype(jnp.float32)
            return 0

        lax.fori_loop(0, _CHUNK // 16, fbody, 0)

        @pl.when(cb == 0)
        def _(ch=ch):
            pltpu.sync_copy(iob.at[pl.ds(0, _CHUNK)],
                            validf_hbm.at[pl.ds(ibase + ch * _CHUNK, _CHUNK)])

    def comp(g, cnt):
        rowv = idxb[pl.ds(cb * _CHUNK + g * 16, 16)]
        vmask = rowv != _ZROW
        plsc.store_compressed(gidxc.at[pl.ds(cnt, 16)], rowv, mask=vmask)
        vr = ibase + cb * _CHUNK + g * 16 + iota16
        plsc.store_compressed(vrowc.at[pl.ds(cnt, 16)], vr, mask=vmask)
        return cnt + plsc.all_reduce_population_count(vmask)[0]

    cnt = lax.fori_loop(0, _CHUNK // 16, comp, jnp.int32(0))
    gidxc[pl.ds(cnt, 16)] = jnp.full((16,), _ZROW, jnp.int32)

    for j in range(3):
        @pl.when(cb == j + 1)
        def _(j=j):
            for ch in range(_NVOX // _CHUNK):
                pltpu.sync_copy(
                    pts_hbm.at[pl.ds(j * _NVOX + ch * _CHUNK, _CHUNK)], zb)

                def pbody(g, _, ch=ch):
                    rowv = idxb[pl.ds(ch * _CHUNK + g * 16, 16)]
                    vf = (rowv != _ZROW).astype(jnp.float32)
                    iob[pl.ds(g * 16, 16)] = zb[pl.ds(g * 16, 16)] * vf
                    return 0

                lax.fori_loop(0, _CHUNK // 16, pbody, 0)
                pltpu.sync_copy(
                    iob.at[pl.ds(0, _CHUNK)],
                    pts3_hbm.at[pl.ds((img * 3 + j) * _NVOX + ch * _CHUNK,
                                      _CHUNK)])

    totpx = nrows0 * _W

    def b1chunk(k, _):
        px0 = (k * 4 + cb) * _PXW
        for c32 in range(4):
            cps = []
            for cc in range(32):
                cidx = c32 * 32 + cc
                cp = pltpu.async_copy(
                    feat_hbm.at[pl.ds((img * _C + cidx) * _HW + px0, _PXW)],
                    src.at[cc], sem)
                cps.append(cp)
            for cp in cps:
                cp.wait()

            def tr(cc, _, c32=c32):
                lane = jnp.full((16,), c32 * 32, jnp.int32) + cc

                def trg(g, _, cc=cc):
                    v = src[cc, pl.ds(g * 16, 16)]
                    plsc.store_scatter(
                        slab, [(g * 16 + iota16) * 128 + lane], v)
                    return 0

                lax.fori_loop(0, _PXW // 16, trg, 0)
                return 0

            lax.fori_loop(0, 32, tr, 0)
        pltpu.sync_copy(slab,
                        t_hbm.at[pl.ds((ib86 + px0) * 128, _PXW * 128)])
        return 0

    nchunk_all = (totpx + _PXW - 1) // _PXW
    nchunk_my = (nchunk_all - cb + 3) // 4
    lax.fori_loop(0, _B1_ON * nchunk_my, b1chunk, 0)

    plsc.subcore_barrier()

    def zinit(g, _):
        zbuf[pl.ds(g * 16, 16)] = jnp.zeros((16,), jnp.float32)
        return 0

    lax.fori_loop(0, _ZB // 16, zinit, 0)
    vb128 = (ibase + cb * _CHUNK) * 128

    def zwr(b, _):
        pltpu.sync_copy(zbuf, vol_hbm.at[pl.ds(vb128 + b * _ZB, _ZB)])
        return 0

    lax.fori_loop(0, _B2_ON * (_CHUNK * 128 // _ZB), zwr, 0)

    def sp(kk, _):
        rv = gidxc[pl.ds(kk * 16, 16)]
        vv = vrowc[pl.ds(kk * 16, 16)]
        for l in range(16):
            row = rv[l]
            vr = vv[l]

            @pl.when(row != _ZROW)
            def _(row=row, vr=vr):
                pltpu.sync_copy(t_hbm.at[pl.ds(row * 128, 128)],
                                vol_hbm.at[pl.ds(vr * 128, 128)])
        return 0

    lax.fori_loop(0, _B2_ON * ((cnt + 15) // 16), sp, 0)


def _sc_gather(feat, dres, idxc, z, pts):
    mesh = plsc.VectorSubcoreMesh(core_axis_name="c", subcore_axis_name="s")
    f = pl.kernel(
        _sc_body,
        out_type=[
            jax.ShapeDtypeStruct((_N_IMG * _NVOX * 128,), jnp.float32),
            jax.ShapeDtypeStruct((_N_IMG * _NVOX,), jnp.float32),
            jax.ShapeDtypeStruct((_N_IMG * 3 * _NVOX,), jnp.float32),
            jax.ShapeDtypeStruct((_TROWS * 128,), jnp.float32),
        ],
        mesh=mesh,
        scratch_types=[
            pltpu.VMEM((_SEC,), jnp.float32),
            pltpu.VMEM((_NVOX,), jnp.int32),
            pltpu.VMEM((_CHUNK,), jnp.float32),
            pltpu.VMEM((_CHUNK,), jnp.float32),
            pltpu.VMEM((32, _PXW), jnp.float32),
            pltpu.VMEM((_PXW * 128,), jnp.float32),
            pltpu.VMEM((_CHUNK + 32,), jnp.int32),
            pltpu.VMEM((_CHUNK + 32,), jnp.int32),
            pltpu.VMEM((_ZB,), jnp.float32),
            pltpu.SemaphoreType.DMA,
        ],
        compiler_params=pltpu.CompilerParams(needs_layout_passes=False),
    )
    vol, validf, pts3, _ = f(feat, dres, idxc, z, pts)
    return vol, validf, pts3


def kernel(features, points, projection, depth, offsets):
    n, C, H, W = features.shape
    nx, ny, nz = points.shape[-3:]

    feat = features.reshape(n * C * H * W)
    dres = jax.image.resize(depth[:, None, :, :], (n, 1, H, W),
                            method="bilinear")[:, 0].reshape(n * H * W)

    off = jnp.tanh(offsets) * MAX_OFFSET
    off = jnp.broadcast_to(off, (n, off.shape[1], 2))
    ptsb = points.reshape(1, 3, _NVOX)
    ptsb = jnp.broadcast_to(ptsb, (n, 3, _NVOX))
    pts_h = jnp.concatenate(
        [ptsb, jnp.ones((n, 1, _NVOX), dtype=ptsb.dtype)], axis=1)
    p23 = jnp.einsum('bij,bjn->bin', projection, pts_h)
    xoff = p23[:, 0] / p23[:, 2] + off[:, :, 0]
    yoff = p23[:, 1] / p23[:, 2] + off[:, :, 1]
    z = p23[:, 2]

    idxc = _tc_prologue(xoff.reshape(_N_IMG * _ROWS, 128),
                        yoff.reshape(_N_IMG * _ROWS, 128),
                        z.reshape(_N_IMG * _ROWS, 128))
    pts = points.reshape(3 * _NVOX)
    vol2d, validf, pts3 = _sc_gather(feat, dres, idxc, z.reshape(-1), pts)

    volume = jnp.transpose(vol2d.reshape(n, nx, ny, nz, C), (0, 4, 1, 2, 3))
    valid_r = (validf > 0).reshape(n, 1, nx, ny, nz)
    pts3 = pts3.reshape(n, 3, nx, ny, nz)
    return volume, valid_r, pts3


# speedup vs baseline: 16.5258x; 1.4527x over previous
import functools

import jax
import jax.numpy as jnp
from jax import lax
from jax.experimental import pallas as pl
from jax.experimental.pallas import tpu as pltpu
from jax.experimental.pallas import tpu_sc as plsc

VOXEL_SIZE_Z = 0.04
MAX_OFFSET = 5.0

_N_IMG = 8
_C = 128
_H = 224
_W = 384
_HW = _H * _W
_NVOX = 25600
_ROWS = 200
_ZROW = _N_IMG * _HW
_TROWS = _ZROW

_NC = 2
_NS = 16

_CHUNK = 6400
_SEC = 32 * _W
_PXW = 256
_ZB = 16384
_B1_ON = 1
_B2_ON = 1


def _tc_prologue_body(x_ref, y_ref, z_ref, idx_ref):
    x = x_ref[...]
    y = y_ref[...]
    z = z_ref[...]
    xi = jnp.round(x).astype(jnp.int32)
    yi = jnp.round(y).astype(jnp.int32)
    valid0 = (xi >= 0) & (yi >= 0) & (xi < _W) & (yi < _H) & (z > 0)
    xc = jnp.clip(xi, 0, _W - 1)
    yc = jnp.clip(yi, 0, _H - 1)
    idx_ref[...] = jnp.where(valid0, yc * _W + xc, -1)


def _tc_prologue(xoff, yoff, z):
    out = pl.pallas_call(
        _tc_prologue_body,
        out_shape=jax.ShapeDtypeStruct((_N_IMG * _ROWS, 128), jnp.int32),
    )(xoff, yoff, z)
    return out.reshape(_N_IMG * _NVOX)


def _reduce_max16(vec):
    return lax.rev(lax.sort(vec, dimension=0), (0,))[0]


def _sc_a_body(d_hbm, idx_hbm, z_hbm, pts_hbm,
               rowptr_hbm, validf_hbm, pts3_hbm,
               plane, idxb, zb, iob):
    s = lax.axis_index("s")
    c = lax.axis_index("c")
    wid = c * _NS + s
    img = wid // 4
    cb = wid % 4

    ibase = pl.multiple_of(img * _NVOX, 8)
    ib86 = pl.multiple_of(img * _HW, 8)

    pltpu.sync_copy(idx_hbm.at[pl.ds(ibase, _NVOX)], idxb)

    def maxbody(g, mv):
        return jnp.maximum(mv, idxb[pl.ds(g * 16, 16)])
    mv16 = lax.fori_loop(0, _NVOX // 16, maxbody,
                         jnp.full((16,), -1, jnp.int32))
    maxv = _reduce_max16(mv16)
    nrows0 = (maxv + _W) // _W
    nsec = (nrows0 + 31) // 32

    for ch in range(_NVOX // _CHUNK):
        pltpu.sync_copy(z_hbm.at[pl.ds(ibase + ch * _CHUNK, _CHUNK)], zb)

        def secbody(sec, _, ch=ch):
            lo = sec * _SEC
            pltpu.sync_copy(d_hbm.at[pl.ds(ib86 + lo, _SEC)], plane)

            def wbody(g, _, ch=ch):
                ic = idxb[pl.ds(ch * _CHUNK + g * 16, 16)]
                insec = (ic >= lo) & (ic < lo + _SEC)
                gi = jnp.clip(ic - lo, 0, _SEC - 1)
                dg = plsc.load_gather(plane, [gi])
                zz = zb[pl.ds(g * 16, 16)]
                win = (zz > dg - jnp.float32(VOXEL_SIZE_Z)) & \
                      (zz < dg + jnp.float32(VOXEL_SIZE_Z))
                conv = insec & win
                newv = jnp.where(conv, -(ib86 + ic) - 2, ic)
                idxb[pl.ds(ch * _CHUNK + g * 16, 16)] = newv
                return 0

            lax.fori_loop(0, _CHUNK // 16, wbody, 0)
            return 0

        lax.fori_loop(0, nsec, secbody, 0)

    iota16 = lax.iota(jnp.int32, 16)
    for ch in range(_NVOX // _CHUNK):
        def fbody(g, _, ch=ch):
            ic = idxb[pl.ds(ch * _CHUNK + g * 16, 16)]
            rowv = jnp.where(ic <= -2, -ic - 2, jnp.int32(_ZROW))
            idxb[pl.ds(ch * _CHUNK + g * 16, 16)] = rowv
            iob[pl.ds(g * 16, 16)] = (rowv != _ZROW).astype(jnp.float32)
            return 0

        lax.fori_loop(0, _CHUNK // 16, fbody, 0)

        @pl.when(cb == 0)
        def _(ch=ch):
            pltpu.sync_copy(iob.at[pl.ds(0, _CHUNK)],
                            validf_hbm.at[pl.ds(ibase + ch * _CHUNK, _CHUNK)])


    for j in range(3):
        @pl.when(cb == j + 1)
        def _(j=j):
            for ch in range(_NVOX // _CHUNK):
                pltpu.sync_copy(
                    pts_hbm.at[pl.ds(j * _NVOX + ch * _CHUNK, _CHUNK)], zb)

                def pbody(g, _, ch=ch):
                    rowv = idxb[pl.ds(ch * _CHUNK + g * 16, 16)]
                    vf = (rowv != _ZROW).astype(jnp.float32)
                    iob[pl.ds(g * 16, 16)] = zb[pl.ds(g * 16, 16)] * vf
                    return 0

                lax.fori_loop(0, _CHUNK // 16, pbody, 0)
                pltpu.sync_copy(
                    iob.at[pl.ds(0, _CHUNK)],
                    pts3_hbm.at[pl.ds((img * 3 + j) * _NVOX + ch * _CHUNK,
                                      _CHUNK)])

    @pl.when(cb == 0)
    def _():
        pltpu.sync_copy(idxb, rowptr_hbm.at[pl.ds(ibase, _NVOX)])


def _sc_b_body(feat_hbm, rowptr_hbm,
               vol_hbm, t_hbm,
               idxb, src, slab, gidxc, vrowc, zbuf, sem):
    s = lax.axis_index("s")
    c = lax.axis_index("c")
    wid = c * _NS + s
    img = wid // 4
    cb = wid % 4

    ibase = pl.multiple_of(img * _NVOX, 8)
    ib86 = pl.multiple_of(img * _HW, 8)
    iota16 = lax.iota(jnp.int32, 16)

    pltpu.sync_copy(rowptr_hbm.at[pl.ds(ibase, _NVOX)], idxb)

    def maxbody(g, mv):
        rv = idxb[pl.ds(g * 16, 16)]
        return jnp.maximum(mv, jnp.where(rv == _ZROW, -1, rv - ib86))
    mv16 = lax.fori_loop(0, _NVOX // 16, maxbody,
                         jnp.full((16,), -1, jnp.int32))
    nrows0 = (_reduce_max16(mv16) + _W) // _W

    def comp(g, cnt):
        rowv = idxb[pl.ds(cb * _CHUNK + g * 16, 16)]
        vmask = rowv != _ZROW
        plsc.store_compressed(gidxc.at[pl.ds(cnt, 16)], rowv, mask=vmask)
        vr = ibase + cb * _CHUNK + g * 16 + iota16
        plsc.store_compressed(vrowc.at[pl.ds(cnt, 16)], vr, mask=vmask)
        return cnt + plsc.all_reduce_population_count(vmask)[0]

    cnt = lax.fori_loop(0, _CHUNK // 16, comp, jnp.int32(0))
    gidxc[pl.ds(cnt, 16)] = jnp.full((16,), _ZROW, jnp.int32)

    totpx = nrows0 * _W

    def b1chunk(k, _):
        px0 = (k * 4 + cb) * _PXW
        for c32 in range(4):
            cps = []
            for cc in range(32):
                cidx = c32 * 32 + cc
                cp = pltpu.async_copy(
                    feat_hbm.at[pl.ds((img * _C + cidx) * _HW + px0, _PXW)],
                    src.at[cc], sem)
                cps.append(cp)
            for cp in cps:
                cp.wait()

            def tr(cc, _, c32=c32):
                lane = jnp.full((16,), c32 * 32, jnp.int32) + cc

                def trg(g, _, cc=cc):
                    v = src[cc, pl.ds(g * 16, 16)]
                    plsc.store_scatter(
                        slab, [(g * 16 + iota16) * 128 + lane], v)
                    return 0

                lax.fori_loop(0, _PXW // 16, trg, 0)
                return 0

            lax.fori_loop(0, 32, tr, 0)
        pltpu.sync_copy(slab,
                        t_hbm.at[pl.ds((ib86 + px0) * 128, _PXW * 128)])
        return 0

    nchunk_all = (totpx + _PXW - 1) // _PXW
    nchunk_my = (nchunk_all - cb + 3) // 4
    lax.fori_loop(0, _B1_ON * nchunk_my, b1chunk, 0)

    plsc.subcore_barrier()

    def zinit(g, _):
        zbuf[pl.ds(g * 16, 16)] = jnp.zeros((16,), jnp.float32)
        return 0

    lax.fori_loop(0, _ZB // 16, zinit, 0)
    vb128 = (ibase + cb * _CHUNK) * 128

    def zwr(b, _):
        pltpu.sync_copy(zbuf, vol_hbm.at[pl.ds(vb128 + b * _ZB, _ZB)])
        return 0

    lax.fori_loop(0, _B2_ON * (_CHUNK * 128 // _ZB), zwr, 0)

    def sp(kk, _):
        rv = gidxc[pl.ds(kk * 16, 16)]
        vv = vrowc[pl.ds(kk * 16, 16)]
        for l in range(16):
            row = rv[l]
            vr = vv[l]

            @pl.when(row != _ZROW)
            def _(row=row, vr=vr):
                pltpu.async_copy(t_hbm.at[pl.ds(row * 128, 128)],
                                 vol_hbm.at[pl.ds(vr * 128, 128)], sem)
        for l in range(16):
            row = rv[l]
            vr = vv[l]

            @pl.when(row != _ZROW)
            def _(row=row, vr=vr):
                pltpu.make_async_copy(
                    t_hbm.at[pl.ds(row * 128, 128)],
                    vol_hbm.at[pl.ds(vr * 128, 128)], sem).wait()
        return 0

    lax.fori_loop(0, _B2_ON * ((cnt + 15) // 16), sp, 0)


def _sc_gather(feat, dres, idxc, z, pts):
    mesh = plsc.VectorSubcoreMesh(core_axis_name="c", subcore_axis_name="s")
    fa = pl.kernel(
        _sc_a_body,
        out_type=[
            jax.ShapeDtypeStruct((_N_IMG * _NVOX,), jnp.int32),
            jax.ShapeDtypeStruct((_N_IMG * _NVOX,), jnp.float32),
            jax.ShapeDtypeStruct((_N_IMG * 3 * _NVOX,), jnp.float32),
        ],
        mesh=mesh,
        scratch_types=[
            pltpu.VMEM((_SEC,), jnp.float32),
            pltpu.VMEM((_NVOX,), jnp.int32),
            pltpu.VMEM((_CHUNK,), jnp.float32),
            pltpu.VMEM((_CHUNK,), jnp.float32),
        ],
        compiler_params=pltpu.CompilerParams(needs_layout_passes=False),
    )
    rowptr, validf, pts3 = fa(dres, idxc, z, pts)

    fb = pl.kernel(
        _sc_b_body,
        out_type=[
            jax.ShapeDtypeStruct((_N_IMG * _NVOX * 128,), jnp.float32),
            jax.ShapeDtypeStruct((_TROWS * 128,), jnp.float32),
        ],
        mesh=mesh,
        scratch_types=[
            pltpu.VMEM((_NVOX,), jnp.int32),
            pltpu.VMEM((32, _PXW), jnp.float32),
            pltpu.VMEM((_PXW * 128,), jnp.float32),
            pltpu.VMEM((_CHUNK + 32,), jnp.int32),
            pltpu.VMEM((_CHUNK + 32,), jnp.int32),
            pltpu.VMEM((_ZB,), jnp.float32),
            pltpu.SemaphoreType.DMA,
        ],
        compiler_params=pltpu.CompilerParams(needs_layout_passes=False),
    )
    vol, _ = fb(feat, rowptr)
    return vol, validf, pts3


def kernel(features, points, projection, depth, offsets):
    n, C, H, W = features.shape
    nx, ny, nz = points.shape[-3:]

    feat = features.reshape(n * C * H * W)
    dres = jax.image.resize(depth[:, None, :, :], (n, 1, H, W),
                            method="bilinear")[:, 0].reshape(n * H * W)

    off = jnp.tanh(offsets) * MAX_OFFSET
    off = jnp.broadcast_to(off, (n, off.shape[1], 2))
    ptsb = points.reshape(1, 3, _NVOX)
    ptsb = jnp.broadcast_to(ptsb, (n, 3, _NVOX))
    pts_h = jnp.concatenate(
        [ptsb, jnp.ones((n, 1, _NVOX), dtype=ptsb.dtype)], axis=1)
    p23 = jnp.einsum('bij,bjn->bin', projection, pts_h)
    xoff = p23[:, 0] / p23[:, 2] + off[:, :, 0]
    yoff = p23[:, 1] / p23[:, 2] + off[:, :, 1]
    z = p23[:, 2]

    idxc = _tc_prologue(xoff.reshape(_N_IMG * _ROWS, 128),
                        yoff.reshape(_N_IMG * _ROWS, 128),
                        z.reshape(_N_IMG * _ROWS, 128))
    pts = points.reshape(3 * _NVOX)
    vol2d, validf, pts3 = _sc_gather(feat, dres, idxc, z.reshape(-1), pts)

    volume = jnp.transpose(vol2d.reshape(n, nx, ny, nz, C), (0, 4, 1, 2, 3))
    valid_r = (validf > 0).reshape(n, 1, nx, ny, nz)
    pts3 = pts3.reshape(n, 3, nx, ny, nz)
    return volume, valid_r, pts3
